# baseline (speedup 1.0000x reference)
import functools

import jax
import jax.numpy as jnp
from jax import lax
from jax.experimental import pallas as pl
from jax.experimental.pallas import tpu as pltpu
from jax.experimental.pallas import tpu_sc as plsc

N = 10000
E = 160000
R = 200
ENT = 256
REL = 128
TIME = 64
IN_MSG = TIME + REL + ENT

NC = 2
NS = 16
CH = 128
CH2 = 64
EP = 163840
KA = 20008
KB = 416
K = 4
F32 = jnp.float32
NPLANES = 8


def _sc_stage1(eq0, eq1, eq2, eq3, rh0, rh1, t64, ones_t, zeros_t,
               srcR, brelR, timeR, keyR, key2R):
    mesh = plsc.VectorSubcoreMesh(core_axis_name="c", subcore_axis_name="s")
    nch = EP // CH // NS
    qtr = nch // 4

    @functools.partial(
        pl.kernel,
        out_type=(
            jax.ShapeDtypeStruct((NPLANES, KA, 64), F32),
            jax.ShapeDtypeStruct((KB, 64), F32),
            jax.ShapeDtypeStruct((KB, 64), F32),
        ),
        mesh=mesh,
        compiler_params=pltpu.CompilerParams(use_tc_tiling_on_sc=False),
        scratch_types=[
            pltpu.VMEM_SHARED((KA, 64), F32),
            pltpu.VMEM_SHARED((KB, 64), F32),
            pltpu.VMEM_SHARED((KB, 64), F32),
            pltpu.VMEM((qtr, CH), jnp.int32),
            pltpu.VMEM((nch, CH), jnp.int32),
            pltpu.VMEM((K, CH, 64), F32),
            pltpu.SemaphoreType.DMA,
            pltpu.SemaphoreType.DMA,
            pltpu.SemaphoreType.DMA,
            pltpu.SemaphoreType.DMA,
            pltpu.SemaphoreType.DMA,
            pltpu.SemaphoreType.DMA,
            pltpu.SemaphoreType.DMA,
            pltpu.SemaphoreType.DMA,
        ],
    )
    def k(eq0_h, eq1_h, eq2_h, eq3_h, rh0_h, rh1_h, t64_h, ones_h, zeros_h,
          src_h, brel_h, time_h, key_h, key2_h, out_h, outT2_h, outC2_h,
          acc, accT2, accC2, idx2d, key2d, rows,
          sg0, sg1, sg2, sg3, ss0, ss1, ss2, ss3):
        core = lax.axis_index("c")
        s = lax.axis_index("s")
        sg = [sg0, sg1, sg2, sg3]
        ss = [ss0, ss1, ss2, ss3]
        pltpu.sync_copy(key_h.at[pl.ds(s * nch, nch)], key2d)

        @pl.when(s == 0)
        def _():
            pltpu.sync_copy(zeros_h.at[pl.ds(0, KB)], accT2)

        @pl.when(s == 1)
        def _():
            pltpu.sync_copy(zeros_h.at[pl.ds(0, KB)], accC2)

        def wait_gather(table, b):
            pltpu.make_async_copy(table.at[idx2d.at[0]], rows.at[b],
                                  sg[b]).wait()

        def wait_scatter(accref, b):
            pltpu.make_async_copy(rows.at[b], accref.at[key2d.at[0]],
                                  ss[b]).wait()

        def run_pass(cid, table, idx_h, accref):
            @pl.when(core == cid)
            def _():
                for h in range(4):
                    pltpu.sync_copy(
                        idx_h.at[pl.ds(s * nch + h * qtr, qtr)], idx2d)
                    for b in range(K):
                        pltpu.async_copy(table.at[idx2d.at[b]], rows.at[b],
                                         sg[b])

                    def it(t, carry, h=h):
                        for b in range(K):
                            li = t * K + b
                            wait_gather(table, b)
                            pltpu.async_copy(
                                rows.at[b], accref.at[key2d.at[h * qtr + li]],
                                ss[b], add=True)
                        for b in range(K):
                            nli = t * K + K + b

                            @pl.when(nli < qtr)
                            def _(nli=nli, b=b):
                                wait_scatter(accref, b)
                                pltpu.async_copy(table.at[idx2d.at[nli]],
                                                 rows.at[b], sg[b])
                        return carry

                    lax.fori_loop(0, qtr // K, it, 0)
                    for b in range(K):
                        wait_scatter(accref, b)

        def run_count_pass(cid, accref):
            @pl.when(core == cid)
            def _():
                pltpu.sync_copy(ones_h, rows.at[0])

                def it(t, carry):
                    ds_ = [
                        pltpu.async_copy(rows.at[0],
                                         accref.at[key2d.at[t * K + b]],
                                         ss[b], add=True)
                        for b in range(K)
                    ]
                    for d in ds_:
                        d.wait()
                    return carry

                lax.fori_loop(0, nch // K, it, 0)

        def restage_key2(cid):
            @pl.when(core == cid)
            def _():
                pltpu.sync_copy(key2_h.at[pl.ds(s * nch, nch)], key2d)

        def flush(cid, plane):
            @pl.when(core == cid)
            def _():
                @pl.when(s < 15)
                def _():
                    pltpu.sync_copy(acc.at[pl.ds(s * 1256, 1256)],
                                    out_h.at[plane, pl.ds(s * 1256, 1256)])

                @pl.when(s == 15)
                def _():
                    pltpu.sync_copy(acc.at[pl.ds(18840, 1168)],
                                    out_h.at[plane, pl.ds(18840, 1168)])

        def zero_acc():
            @pl.when(s < 15)
            def _():
                pltpu.sync_copy(zeros_h, acc.at[pl.ds(s * 1256, 1256)])

            @pl.when(s == 15)
            def _():
                pltpu.sync_copy(zeros_h.at[pl.ds(0, 1168)],
                                acc.at[pl.ds(18840, 1168)])

        rounds = [
            ((eq0_h, src_h, 2), (eq1_h, src_h, 3)),
            ((eq2_h, src_h, 4), (eq3_h, src_h, 5)),
            ((rh0_h, brel_h, 0), (rh1_h, brel_h, 1)),
        ]
        for p0, p1 in rounds:
            zero_acc()
            plsc.subcore_barrier()
            run_pass(0, p0[0], p0[1], acc)
            run_pass(1, p1[0], p1[1], acc)
            plsc.subcore_barrier()
            flush(0, p0[2])
            flush(1, p1[2])
            plsc.subcore_barrier()

        zero_acc()
        plsc.subcore_barrier()
        run_pass(0, t64_h, time_h, acc)
        run_count_pass(1, acc)
        restage_key2(0)
        restage_key2(1)
        run_count_pass(0, accC2)
        run_pass(1, t64_h, time_h, accT2)
        plsc.subcore_barrier()
        flush(0, 6)
        flush(1, 7)

        @pl.when((core == 1) & (s < 13))
        def _():
            pltpu.sync_copy(accT2.at[pl.ds(s * 32, 32)],
                            outT2_h.at[pl.ds(s * 32, 32)])

        @pl.when((core == 0) & (s < 13))
        def _():
            pltpu.sync_copy(accC2.at[pl.ds(s * 32, 32)],
                            outC2_h.at[pl.ds(s * 32, 32)])

    return k(eq0, eq1, eq2, eq3, rh0, rh1, t64, ones_t, zeros_t,
             srcR, brelR, timeR, keyR, key2R)


def _sc_stage2(ent_new, zerosS, srcR2, dstR2, keyR2):
    mesh = plsc.VectorSubcoreMesh(core_axis_name="c", subcore_axis_name="s")
    nch = EP // CH2 // (NC * NS)

    @functools.partial(
        pl.kernel,
        out_type=(
            jax.ShapeDtypeStruct((NC, KB, ENT), F32),
            jax.ShapeDtypeStruct((NC, KB, ENT), F32),
        ),
        mesh=mesh,
        compiler_params=pltpu.CompilerParams(use_tc_tiling_on_sc=False),
        scratch_types=[
            pltpu.VMEM_SHARED((KB, ENT), F32),
            pltpu.VMEM_SHARED((KB, ENT), F32),
            pltpu.VMEM((nch, CH2), jnp.int32),
            pltpu.VMEM((nch, CH2), jnp.int32),
            pltpu.VMEM((nch, CH2), jnp.int32),
            pltpu.VMEM((2, CH2, ENT), F32),
            pltpu.VMEM((2, CH2, ENT), F32),
            pltpu.SemaphoreType.DMA,
            pltpu.SemaphoreType.DMA,
            pltpu.SemaphoreType.DMA,
            pltpu.SemaphoreType.DMA,
            pltpu.SemaphoreType.DMA,
            pltpu.SemaphoreType.DMA,
            pltpu.SemaphoreType.DMA,
            pltpu.SemaphoreType.DMA,
        ],
    )
    def k(ent_h, zS_h, src_h, dst_h, key_h,
          outS_h, outD_h,
          accS, accD, src2d, dst2d, key2d, rs, rd,
          gs0, gs1, gd0, gd1, ws0, ws1, wd0, wd1):
        core = lax.axis_index("c")
        s = lax.axis_index("s")
        gs = [gs0, gs1]
        gd = [gd0, gd1]
        ws = [ws0, ws1]
        wd = [wd0, wd1]
        wid = s * NC + core
        r0 = wid * nch
        pltpu.sync_copy(src_h.at[pl.ds(r0, nch)], src2d)
        pltpu.sync_copy(dst_h.at[pl.ds(r0, nch)], dst2d)
        pltpu.sync_copy(key_h.at[pl.ds(r0, nch)], key2d)

        @pl.when(s == 0)
        def _():
            pltpu.sync_copy(zS_h, accS)

        @pl.when(s == 1)
        def _():
            pltpu.sync_copy(zS_h, accD)

        plsc.subcore_barrier()

        def issue_gathers(i, sl):
            pltpu.async_copy(ent_h.at[src2d.at[i]], rs.at[sl], gs[sl])
            pltpu.async_copy(ent_h.at[dst2d.at[i]], rd.at[sl], gd[sl])

        def wait_gathers(sl):
            pltpu.make_async_copy(ent_h.at[src2d.at[0]], rs.at[sl], gs[sl]).wait()
            pltpu.make_async_copy(ent_h.at[dst2d.at[0]], rd.at[sl], gd[sl]).wait()

        def issue_scatters(i, sl):
            key = key2d.at[i]
            pltpu.async_copy(rs.at[sl], accS.at[key], ws[sl], add=True)
            pltpu.async_copy(rd.at[sl], accD.at[key], wd[sl], add=True)

        def wait_scatters(sl):
            pltpu.make_async_copy(rs.at[sl], accS.at[key2d.at[0]], ws[sl]).wait()
            pltpu.make_async_copy(rd.at[sl], accD.at[key2d.at[0]], wd[sl]).wait()

        issue_gathers(0, 0)
        issue_gathers(1, 1)

        def it(u, carry):
            for sl in range(2):
                i = 2 * u + sl
                wait_gathers(sl)
                issue_scatters(i, sl)
            for sl in range(2):
                ni = 2 * u + 2 + sl

                @pl.when(ni < nch)
                def _(ni=ni, sl=sl):
                    wait_scatters(sl)
                    issue_gathers(ni, sl)
            return carry

        lax.fori_loop(0, nch // 2, it, 0)
        wait_scatters(0)
        wait_scatters(1)
        plsc.subcore_barrier()

        @pl.when(s < 13)
        def _():
            nr = 32
            f0 = s * nr
            pltpu.sync_copy(accS.at[pl.ds(f0, nr)], outS_h.at[core, pl.ds(f0, nr)])
            pltpu.sync_copy(accD.at[pl.ds(f0, nr)], outD_h.at[core, pl.ds(f0, nr)])

    return k(ent_new, zerosS, srcR2, dstR2, keyR2)


def _tc_stage1(SA, ent_emb, wIt, wOt, wSt, bias3):
    BM = 1000
    nb = N // BM

    def body(s0_ref, s1_ref, e_ref, wI_ref, wO_ref, wS_ref, b_ref, o_ref):
        dot = functools.partial(jnp.dot, preferred_element_type=F32,
                                precision=lax.Precision.HIGHEST)
        blk0 = s0_ref[...]
        blk1 = s1_ref[...]
        s0 = jnp.concatenate([blk0[p] for p in range(7)], axis=1)
        d0 = blk0[7][:, 0:1]
        s1 = jnp.concatenate([blk1[p] for p in range(7)], axis=1)
        d1 = blk1[7][:, 0:1]
        m = (dot(s0, wI_ref[...]) + d0 * b_ref[0:1, :]
             + dot(s1, wO_ref[...]) + d1 * b_ref[1:2, :])
        h = m / jnp.maximum(d0 + d1, 1.0)
        o_ref[...] = dot(e_ref[...], wS_ref[...]) + b_ref[2:3, :] + h

    return pl.pallas_call(
        body,
        grid=(nb,),
        in_specs=[
            pl.BlockSpec((NPLANES, BM, 64), lambda i: (0, i, 0)),
            pl.BlockSpec((NPLANES, BM, 64), lambda i: (0, i + nb, 0)),
            pl.BlockSpec((BM, ENT), lambda i: (i, 0)),
            pl.BlockSpec((IN_MSG, ENT), lambda i: (0, 0)),
            pl.BlockSpec((IN_MSG, ENT), lambda i: (0, 0)),
            pl.BlockSpec((ENT, ENT), lambda i: (0, 0)),
            pl.BlockSpec((8, ENT), lambda i: (0, 0)),
        ],
        out_specs=pl.BlockSpec((BM, ENT), lambda i: (i, 0)),
        out_shape=jax.ShapeDtypeStruct((N, ENT), F32),
    )(SA, SA, ent_emb, wIt, wOt, wSt, bias3)


def _tc_stage2(outS, outD, outT2, outC2, rel_emb, wot, wit, wrt, bias3r):
    def body(S_ref, D_ref, T_ref, C_ref, rel_ref, wo_ref, wi_ref, wr_ref,
             b_ref, o_ref):
        dot = functools.partial(jnp.dot, preferred_element_type=F32,
                                precision=lax.Precision.HIGHEST)
        US = S_ref[0] + S_ref[1]
        UD = D_ref[0] + D_ref[1]
        UT = T_ref[...]
        c = C_ref[...][:, 0:1]
        p0 = (dot(US[0:R], wo_ref[0:ENT]) + dot(UD[0:R], wo_ref[ENT:2 * ENT])
              + dot(UT[0:R], wo_ref[2 * ENT:2 * ENT + TIME])
              + c[0:R] * b_ref[0:1, :])
        p1 = (dot(US[R:2 * R], wi_ref[0:ENT])
              + dot(UD[R:2 * R], wi_ref[ENT:2 * ENT])
              + dot(UT[R:2 * R], wi_ref[2 * ENT:2 * ENT + TIME])
              + c[R:2 * R] * b_ref[1:2, :])
        cnt = c[0:R] + c[R:2 * R]
        h = (p0 + p1) / jnp.maximum(cnt, 1.0)
        o_ref[...] = dot(rel_ref[...], wr_ref[...]) + b_ref[2:3, :] + h

    return pl.pallas_call(
        body,
        out_shape=jax.ShapeDtypeStruct((R, REL), F32),
    )(outS, outD, outT2, outC2, rel_emb, wot, wit, wrt, bias3r)


def kernel(ent_emb, rel_emb, time_emb, edge_index, b_rel, time_idx, inv,
           W_I_w, W_I_b, W_O_w, W_O_b, W_S_w, W_S_b,
           W_r_ori_w, W_r_ori_b, W_r_inv_w, W_r_inv_b, W_R_w, W_R_b):
    i32 = jnp.int32
    pad = EP - E
    src = edge_index[0].astype(i32)
    dst = edge_index[1].astype(i32)
    zpad = jnp.zeros((pad,), i32)
    srcP = jnp.concatenate([src, zpad])
    dstP = jnp.concatenate([dst, zpad])
    brelP = jnp.concatenate([b_rel.astype(i32), zpad])
    timeP = jnp.concatenate([time_idx.astype(i32), zpad])
    invP = jnp.concatenate([inv.astype(i32), jnp.full((pad,), 2, i32)])
    nchT = EP // CH
    nchT2 = EP // CH2
    srcR = srcP.reshape(nchT, CH)
    brelR = brelP.reshape(nchT, CH)
    timeR = timeP.reshape(nchT, CH)
    keyR = (invP * N + dstP).reshape(nchT, CH)
    key2R = (invP * R + brelP).reshape(nchT, CH)
    srcR2 = srcP.reshape(nchT2, CH2)
    dstR2 = dstP.reshape(nchT2, CH2)
    keyR2 = (invP * R + brelP).reshape(nchT2, CH2)

    eq = [ent_emb[:, 64 * k:64 * (k + 1)] for k in range(4)]
    rh0 = rel_emb[:, :64]
    rh1 = rel_emb[:, 64:]
    ones_t = jnp.ones((CH, 64), F32)
    zeros_t = jnp.zeros((1256, 64), F32)

    SA, outT2, outC2 = _sc_stage1(
        eq[0], eq[1], eq[2], eq[3], rh0, rh1, time_emb,
        ones_t, zeros_t, srcR, brelR, timeR, keyR, key2R)

    bias3 = jnp.concatenate(
        [W_I_b[None], W_O_b[None], W_S_b[None], jnp.zeros((5, ENT), F32)], 0)
    ent_new = _tc_stage1(SA, ent_emb, W_I_w.T, W_O_w.T, W_S_w.T, bias3)

    zerosS = jnp.zeros((KB, ENT), F32)
    outS, outD = _sc_stage2(ent_new, zerosS, srcR2, dstR2, keyR2)

    bias3r = jnp.concatenate(
        [W_r_ori_b[None], W_r_inv_b[None], W_R_b[None], jnp.zeros((5, REL), F32)], 0)
    rel_new = _tc_stage2(outS, outD, outT2, outC2, rel_emb,
                         W_r_ori_w.T, W_r_inv_w.T, W_R_w.T, bias3r)
    return ent_new, rel_new


# speedup vs baseline: 1.0520x; 1.0520x over previous
import functools

import jax
import jax.numpy as jnp
from jax import lax
from jax.experimental import pallas as pl
from jax.experimental.pallas import tpu as pltpu
from jax.experimental.pallas import tpu_sc as plsc

N = 10000
E = 160000
R = 200
ENT = 256
REL = 128
TIME = 64
IN_MSG = TIME + REL + ENT

NC = 2
NS = 16
CH = 128
CH2 = 64
EP = 163840
KA = 20008
KB = 416
K = 4
F32 = jnp.float32
NPLANES = 8


def _sc_stage1(eq0, eq1, eq2, eq3, rh0, rh1, t64, ones_t, zeros_t,
               srcR, brelR, timeR, keyR):
    mesh = plsc.VectorSubcoreMesh(core_axis_name="c", subcore_axis_name="s")
    nch = EP // CH // NS
    half = nch // 2

    @functools.partial(
        pl.kernel,
        out_type=jax.ShapeDtypeStruct((NPLANES, KA, 64), F32),
        mesh=mesh,
        compiler_params=pltpu.CompilerParams(use_tc_tiling_on_sc=False),
        scratch_types=[
            pltpu.VMEM_SHARED((KA, 64), F32),
            pltpu.VMEM((half, CH), jnp.int32),
            pltpu.VMEM((nch, CH), jnp.int32),
            pltpu.VMEM((K, CH, 64), F32),
            pltpu.SemaphoreType.DMA,
            pltpu.SemaphoreType.DMA,
            pltpu.SemaphoreType.DMA,
            pltpu.SemaphoreType.DMA,
            pltpu.SemaphoreType.DMA,
            pltpu.SemaphoreType.DMA,
            pltpu.SemaphoreType.DMA,
            pltpu.SemaphoreType.DMA,
        ],
    )
    def k(eq0_h, eq1_h, eq2_h, eq3_h, rh0_h, rh1_h, t64_h, ones_h, zeros_h,
          src_h, brel_h, time_h, key_h, out_h,
          acc, idx2d, key2d, rows,
          sg0, sg1, sg2, sg3, ss0, ss1, ss2, ss3):
        core = lax.axis_index("c")
        s = lax.axis_index("s")
        sg = [sg0, sg1, sg2, sg3]
        ss = [ss0, ss1, ss2, ss3]
        pltpu.sync_copy(key_h.at[pl.ds(s * nch, nch)], key2d)

        def wait_gather(table, b):
            pltpu.make_async_copy(table.at[idx2d.at[0]], rows.at[b],
                                  sg[b]).wait()

        def wait_scatter(b):
            pltpu.make_async_copy(rows.at[b], acc.at[key2d.at[0]],
                                  ss[b]).wait()

        def run_pass(cid, table, idx_h):
            @pl.when(core == cid)
            def _():
                for h in range(2):
                    pltpu.sync_copy(
                        idx_h.at[pl.ds(s * nch + h * half, half)], idx2d)
                    for b in range(K):
                        pltpu.async_copy(table.at[idx2d.at[b]], rows.at[b],
                                         sg[b])

                    def it(t, carry, h=h):
                        for b in range(K):
                            li = t * K + b
                            wait_gather(table, b)
                            pltpu.async_copy(
                                rows.at[b], acc.at[key2d.at[h * half + li]],
                                ss[b], add=True)
                        for b in range(K):
                            nli = t * K + K + b

                            @pl.when(nli < half)
                            def _(nli=nli, b=b):
                                wait_scatter(b)
                                pltpu.async_copy(table.at[idx2d.at[nli]],
                                                 rows.at[b], sg[b])
                        return carry

                    lax.fori_loop(0, half // K, it, 0)
                    for b in range(K):
                        wait_scatter(b)

        def run_count_pass(cid):
            @pl.when(core == cid)
            def _():
                pltpu.sync_copy(ones_h, rows.at[0])

                def it(t, carry):
                    ds_ = [
                        pltpu.async_copy(rows.at[0],
                                         acc.at[key2d.at[t * K + b]],
                                         ss[b], add=True)
                        for b in range(K)
                    ]
                    for d in ds_:
                        d.wait()
                    return carry

                lax.fori_loop(0, nch // K, it, 0)

        def flush(cid, plane):
            @pl.when(core == cid)
            def _():
                @pl.when(s < 15)
                def _():
                    pltpu.sync_copy(acc.at[pl.ds(s * 1256, 1256)],
                                    out_h.at[plane, pl.ds(s * 1256, 1256)])

                @pl.when(s == 15)
                def _():
                    pltpu.sync_copy(acc.at[pl.ds(18840, 1168)],
                                    out_h.at[plane, pl.ds(18840, 1168)])

        def zero_acc():
            @pl.when(s < 15)
            def _():
                pltpu.sync_copy(zeros_h, acc.at[pl.ds(s * 1256, 1256)])

            @pl.when(s == 15)
            def _():
                pltpu.sync_copy(zeros_h.at[pl.ds(0, 1168)],
                                acc.at[pl.ds(18840, 1168)])

        rounds = [
            ((eq0_h, src_h, 2), (eq1_h, src_h, 3)),
            ((eq2_h, src_h, 4), (eq3_h, src_h, 5)),
            ((rh0_h, brel_h, 0), (rh1_h, brel_h, 1)),
            ((t64_h, time_h, 6), (None, None, 7)),
        ]
        for p0, p1 in rounds:
            zero_acc()
            plsc.subcore_barrier()
            run_pass(0, p0[0], p0[1])
            if p1[0] is None:
                run_count_pass(1)
            else:
                run_pass(1, p1[0], p1[1])
            plsc.subcore_barrier()
            flush(0, p0[2])
            flush(1, p1[2])
            plsc.subcore_barrier()

    return k(eq0, eq1, eq2, eq3, rh0, rh1, t64, ones_t, zeros_t,
             srcR, brelR, timeR, keyR)


def _sc_stage2(ent_new, t64, ones_t, zerosS, zerosT,
               srcR2, dstR2, timeR2, keyR2):
    mesh = plsc.VectorSubcoreMesh(core_axis_name="c", subcore_axis_name="s")
    nch = EP // CH2 // (NC * NS)

    @functools.partial(
        pl.kernel,
        out_type=(
            jax.ShapeDtypeStruct((NC, 2, KB, ENT), F32),
            jax.ShapeDtypeStruct((NC, 2, KB, ENT), F32),
            jax.ShapeDtypeStruct((NC, KB, 64), F32),
            jax.ShapeDtypeStruct((NC, KB, 64), F32),
        ),
        mesh=mesh,
        compiler_params=pltpu.CompilerParams(use_tc_tiling_on_sc=False),
        scratch_types=[
            pltpu.VMEM_SHARED((2, KB, ENT), F32),
            pltpu.VMEM_SHARED((2, KB, ENT), F32),
            pltpu.VMEM_SHARED((KB, 64), F32),
            pltpu.VMEM_SHARED((KB, 64), F32),
            pltpu.VMEM((nch, CH2), jnp.int32),
            pltpu.VMEM((nch, CH2), jnp.int32),
            pltpu.VMEM((nch, CH2), jnp.int32),
            pltpu.VMEM((nch, CH2), jnp.int32),
            pltpu.VMEM((2, CH2, ENT), F32),
            pltpu.VMEM((2, CH2, ENT), F32),
            pltpu.VMEM((2, CH2, 64), F32),
            pltpu.VMEM((CH2, 64), F32),
            pltpu.SemaphoreType.DMA,
            pltpu.SemaphoreType.DMA,
            pltpu.SemaphoreType.DMA,
            pltpu.SemaphoreType.DMA,
            pltpu.SemaphoreType.DMA,
            pltpu.SemaphoreType.DMA,
            pltpu.SemaphoreType.DMA,
            pltpu.SemaphoreType.DMA,
            pltpu.SemaphoreType.DMA,
            pltpu.SemaphoreType.DMA,
            pltpu.SemaphoreType.DMA,
            pltpu.SemaphoreType.DMA,
            pltpu.SemaphoreType.DMA,
            pltpu.SemaphoreType.DMA,
        ],
    )
    def k(ent_h, t64_h, ones_h, zS_h, zT_h, src_h, dst_h, time_h, key_h,
          outS_h, outD_h, outT_h, outC_h,
          accS, accD, accT, accC, src2d, dst2d, time2d, key2d,
          rs, rd, rt, rones,
          gs0, gs1, gd0, gd1, gt0, gt1,
          ws0, ws1, wd0, wd1, wt0, wt1, wc0, wc1):
        core = lax.axis_index("c")
        s = lax.axis_index("s")
        gs = [gs0, gs1]
        gd = [gd0, gd1]
        gt = [gt0, gt1]
        ws = [ws0, ws1]
        wd = [wd0, wd1]
        wt = [wt0, wt1]
        wc = [wc0, wc1]
        pltpu.sync_copy(ones_h, rones)
        wid = s * NC + core
        r0 = wid * nch
        pltpu.sync_copy(src_h.at[pl.ds(r0, nch)], src2d)
        pltpu.sync_copy(dst_h.at[pl.ds(r0, nch)], dst2d)
        pltpu.sync_copy(time_h.at[pl.ds(r0, nch)], time2d)
        pltpu.sync_copy(key_h.at[pl.ds(r0, nch)], key2d)

        @pl.when(s == 0)
        def _():
            pltpu.sync_copy(zS_h, accS.at[0])

        @pl.when(s == 1)
        def _():
            pltpu.sync_copy(zS_h, accS.at[1])

        @pl.when(s == 2)
        def _():
            pltpu.sync_copy(zS_h, accD.at[0])

        @pl.when(s == 3)
        def _():
            pltpu.sync_copy(zS_h, accD.at[1])

        @pl.when(s == 4)
        def _():
            pltpu.sync_copy(zT_h, accT)

        @pl.when(s == 5)
        def _():
            pltpu.sync_copy(zT_h, accC)

        plsc.subcore_barrier()

        def issue_gathers(i, sl):
            pltpu.async_copy(ent_h.at[src2d.at[i]], rs.at[sl], gs[sl])
            pltpu.async_copy(ent_h.at[dst2d.at[i]], rd.at[sl], gd[sl])
            pltpu.async_copy(t64_h.at[time2d.at[i]], rt.at[sl], gt[sl])

        def wait_gathers(sl):
            pltpu.make_async_copy(ent_h.at[src2d.at[0]], rs.at[sl], gs[sl]).wait()
            pltpu.make_async_copy(ent_h.at[dst2d.at[0]], rd.at[sl], gd[sl]).wait()
            pltpu.make_async_copy(t64_h.at[time2d.at[0]], rt.at[sl], gt[sl]).wait()

        def issue_scatters(i, sl):
            key = key2d.at[i]
            pltpu.async_copy(rs.at[sl], accS.at[sl].at[key], ws[sl], add=True)
            pltpu.async_copy(rd.at[sl], accD.at[sl].at[key], wd[sl], add=True)
            pltpu.async_copy(rt.at[sl], accT.at[key], wt[sl], add=True)
            pltpu.async_copy(rones, accC.at[key], wc[sl], add=True)

        def wait_scatters(sl):
            pltpu.make_async_copy(rs.at[sl], accS.at[sl].at[key2d.at[0]], ws[sl]).wait()
            pltpu.make_async_copy(rd.at[sl], accD.at[sl].at[key2d.at[0]], wd[sl]).wait()
            pltpu.make_async_copy(rt.at[sl], accT.at[key2d.at[0]], wt[sl]).wait()
            pltpu.make_async_copy(rones, accC.at[key2d.at[0]], wc[sl]).wait()

        issue_gathers(0, 0)
        issue_gathers(1, 1)

        def it(u, carry):
            for sl in range(2):
                i = 2 * u + sl
                wait_gathers(sl)
                issue_scatters(i, sl)
            for sl in range(2):
                ni = 2 * u + 2 + sl

                @pl.when(ni < nch)
                def _(ni=ni, sl=sl):
                    wait_scatters(sl)
                    issue_gathers(ni, sl)
            return carry

        lax.fori_loop(0, nch // 2, it, 0)
        wait_scatters(0)
        wait_scatters(1)
        plsc.subcore_barrier()

        @pl.when(s < 13)
        def _():
            nr = 32
            f0 = s * nr
            for sl in range(2):
                pltpu.sync_copy(accS.at[sl, pl.ds(f0, nr)],
                                outS_h.at[core, sl, pl.ds(f0, nr)])
                pltpu.sync_copy(accD.at[sl, pl.ds(f0, nr)],
                                outD_h.at[core, sl, pl.ds(f0, nr)])
            pltpu.sync_copy(accT.at[pl.ds(f0, nr)], outT_h.at[core, pl.ds(f0, nr)])
            pltpu.sync_copy(accC.at[pl.ds(f0, nr)], outC_h.at[core, pl.ds(f0, nr)])

    return k(ent_new, t64, ones_t, zerosS, zerosT, srcR2, dstR2, timeR2, keyR2)


def _tc_stage1(SA, ent_emb, wIt, wOt, wSt, bias3):
    BM = 1000
    nb = N // BM

    def body(s0_ref, s1_ref, e_ref, wI_ref, wO_ref, wS_ref, b_ref, o_ref):
        dot = functools.partial(jnp.dot, preferred_element_type=F32,
                                precision=lax.Precision.HIGHEST)
        blk0 = s0_ref[...]
        blk1 = s1_ref[...]
        s0 = jnp.concatenate([blk0[p] for p in range(7)], axis=1)
        d0 = blk0[7][:, 0:1]
        s1 = jnp.concatenate([blk1[p] for p in range(7)], axis=1)
        d1 = blk1[7][:, 0:1]
        m = (dot(s0, wI_ref[...]) + d0 * b_ref[0:1, :]
             + dot(s1, wO_ref[...]) + d1 * b_ref[1:2, :])
        h = m / jnp.maximum(d0 + d1, 1.0)
        o_ref[...] = dot(e_ref[...], wS_ref[...]) + b_ref[2:3, :] + h

    return pl.pallas_call(
        body,
        grid=(nb,),
        in_specs=[
            pl.BlockSpec((NPLANES, BM, 64), lambda i: (0, i, 0)),
            pl.BlockSpec((NPLANES, BM, 64), lambda i: (0, i + nb, 0)),
            pl.BlockSpec((BM, ENT), lambda i: (i, 0)),
            pl.BlockSpec((IN_MSG, ENT), lambda i: (0, 0)),
            pl.BlockSpec((IN_MSG, ENT), lambda i: (0, 0)),
            pl.BlockSpec((ENT, ENT), lambda i: (0, 0)),
            pl.BlockSpec((8, ENT), lambda i: (0, 0)),
        ],
        out_specs=pl.BlockSpec((BM, ENT), lambda i: (i, 0)),
        out_shape=jax.ShapeDtypeStruct((N, ENT), F32),
    )(SA, SA, ent_emb, wIt, wOt, wSt, bias3)


def _tc_stage2(outS, outD, outT, outC, rel_emb, wot, wit, wrt, bias3r):
    def body(S_ref, D_ref, T_ref, C_ref, rel_ref, wo_ref, wi_ref, wr_ref,
             b_ref, o_ref):
        dot = functools.partial(jnp.dot, preferred_element_type=F32,
                                precision=lax.Precision.HIGHEST)
        US = S_ref[0, 0] + S_ref[0, 1] + S_ref[1, 0] + S_ref[1, 1]
        UD = D_ref[0, 0] + D_ref[0, 1] + D_ref[1, 0] + D_ref[1, 1]
        UT = T_ref[0] + T_ref[1]
        Cc = C_ref[0] + C_ref[1]
        c = Cc[:, 0:1]
        p0 = (dot(US[0:R], wo_ref[0:ENT]) + dot(UD[0:R], wo_ref[ENT:2 * ENT])
              + dot(UT[0:R], wo_ref[2 * ENT:2 * ENT + TIME])
              + c[0:R] * b_ref[0:1, :])
        p1 = (dot(US[R:2 * R], wi_ref[0:ENT])
              + dot(UD[R:2 * R], wi_ref[ENT:2 * ENT])
              + dot(UT[R:2 * R], wi_ref[2 * ENT:2 * ENT + TIME])
              + c[R:2 * R] * b_ref[1:2, :])
        cnt = c[0:R] + c[R:2 * R]
        h = (p0 + p1) / jnp.maximum(cnt, 1.0)
        o_ref[...] = dot(rel_ref[...], wr_ref[...]) + b_ref[2:3, :] + h

    return pl.pallas_call(
        body,
        out_shape=jax.ShapeDtypeStruct((R, REL), F32),
    )(outS, outD, outT, outC, rel_emb, wot, wit, wrt, bias3r)


def kernel(ent_emb, rel_emb, time_emb, edge_index, b_rel, time_idx, inv,
           W_I_w, W_I_b, W_O_w, W_O_b, W_S_w, W_S_b,
           W_r_ori_w, W_r_ori_b, W_r_inv_w, W_r_inv_b, W_R_w, W_R_b):
    i32 = jnp.int32
    pad = EP - E
    src = edge_index[0].astype(i32)
    dst = edge_index[1].astype(i32)
    zpad = jnp.zeros((pad,), i32)
    srcP = jnp.concatenate([src, zpad])
    dstP = jnp.concatenate([dst, zpad])
    brelP = jnp.concatenate([b_rel.astype(i32), zpad])
    timeP = jnp.concatenate([time_idx.astype(i32), zpad])
    invP = jnp.concatenate([inv.astype(i32), jnp.full((pad,), 2, i32)])
    nchT = EP // CH
    nchT2 = EP // CH2
    srcR = srcP.reshape(nchT, CH)
    brelR = brelP.reshape(nchT, CH)
    timeR = timeP.reshape(nchT, CH)
    keyR = (invP * N + dstP).reshape(nchT, CH)
    srcR2 = srcP.reshape(nchT2, CH2)
    dstR2 = dstP.reshape(nchT2, CH2)
    timeR2 = timeP.reshape(nchT2, CH2)
    keyR2 = (invP * R + brelP).reshape(nchT2, CH2)

    eq = [ent_emb[:, 64 * k:64 * (k + 1)] for k in range(4)]
    rh0 = rel_emb[:, :64]
    rh1 = rel_emb[:, 64:]
    ones_t = jnp.ones((CH, 64), F32)
    zeros_t = jnp.zeros((1256, 64), F32)

    SA = _sc_stage1(eq[0], eq[1], eq[2], eq[3], rh0, rh1, time_emb,
                    ones_t, zeros_t, srcR, brelR, timeR, keyR)

    bias3 = jnp.concatenate(
        [W_I_b[None], W_O_b[None], W_S_b[None], jnp.zeros((5, ENT), F32)], 0)
    ent_new = _tc_stage1(SA, ent_emb, W_I_w.T, W_O_w.T, W_S_w.T, bias3)

    ones_t2 = jnp.ones((CH2, 64), F32)
    zerosS = jnp.zeros((KB, ENT), F32)
    zerosT = jnp.zeros((KB, 64), F32)
    outS, outD, outT, outC = _sc_stage2(
        ent_new, time_emb, ones_t2, zerosS, zerosT,
        srcR2, dstR2, timeR2, keyR2)

    bias3r = jnp.concatenate(
        [W_r_ori_b[None], W_r_inv_b[None], W_R_b[None], jnp.zeros((5, REL), F32)], 0)
    rel_new = _tc_stage2(outS, outD, outT, outC, rel_emb,
                         W_r_ori_w.T, W_r_inv_w.T, W_R_w.T, bias3r)
    return ent_new, rel_new


# speedup vs baseline: 1.1137x; 1.0587x over previous
import functools

import jax
import jax.numpy as jnp
from jax import lax
from jax.experimental import pallas as pl
from jax.experimental.pallas import tpu as pltpu
from jax.experimental.pallas import tpu_sc as plsc

N = 10000
E = 160000
R = 200
ENT = 256
REL = 128
TIME = 64
IN_MSG = TIME + REL + ENT

NC = 2
NS = 16
CH = 128
CH2 = 64
EP = 163840
KA = 20008
KB = 416
K = 4
F32 = jnp.float32
NPLANES = 8


def _sc_stage1(eq0, eq1, eq2, eq3, rh0, rh1, t64, ones_t, zeros_t,
               srcR, brelR, timeR, keyR):
    mesh = plsc.VectorSubcoreMesh(core_axis_name="c", subcore_axis_name="s")
    nch = EP // CH // NS
    half = nch // 2

    @functools.partial(
        pl.kernel,
        out_type=jax.ShapeDtypeStruct((NPLANES, KA, 64), F32),
        mesh=mesh,
        compiler_params=pltpu.CompilerParams(use_tc_tiling_on_sc=False),
        scratch_types=[
            pltpu.VMEM_SHARED((KA, 64), F32),
            pltpu.VMEM((half, CH), jnp.int32),
            pltpu.VMEM((nch, CH), jnp.int32),
            pltpu.VMEM((K, CH, 64), F32),
            pltpu.SemaphoreType.DMA,
            pltpu.SemaphoreType.DMA,
            pltpu.SemaphoreType.DMA,
            pltpu.SemaphoreType.DMA,
            pltpu.SemaphoreType.DMA,
            pltpu.SemaphoreType.DMA,
            pltpu.SemaphoreType.DMA,
            pltpu.SemaphoreType.DMA,
        ],
    )
    def k(eq0_h, eq1_h, eq2_h, eq3_h, rh0_h, rh1_h, t64_h, ones_h, zeros_h,
          src_h, brel_h, time_h, key_h, out_h,
          acc, idx2d, key2d, rows,
          sg0, sg1, sg2, sg3, ss0, ss1, ss2, ss3):
        core = lax.axis_index("c")
        s = lax.axis_index("s")
        sg = [sg0, sg1, sg2, sg3]
        ss = [ss0, ss1, ss2, ss3]
        pltpu.sync_copy(key_h.at[pl.ds(s * nch, nch)], key2d)

        def wait_gather(table, b):
            pltpu.make_async_copy(table.at[idx2d.at[0]], rows.at[b],
                                  sg[b]).wait()

        def wait_scatter(b):
            pltpu.make_async_copy(rows.at[b], acc.at[key2d.at[0]],
                                  ss[b]).wait()

        def run_pass(cid, table, idx_h):
            @pl.when(core == cid)
            def _():
                for h in range(2):
                    pltpu.sync_copy(
                        idx_h.at[pl.ds(s * nch + h * half, half)], idx2d)
                    for b in range(K):
                        pltpu.async_copy(table.at[idx2d.at[b]], rows.at[b],
                                         sg[b])

                    def it(t, carry, h=h):
                        for b in range(K):
                            li = t * K + b
                            wait_gather(table, b)
                            pltpu.async_copy(
                                rows.at[b], acc.at[key2d.at[h * half + li]],
                                ss[b], add=True)
                        for b in range(K):
                            nli = t * K + K + b

                            @pl.when(nli < half)
                            def _(nli=nli, b=b):
                                wait_scatter(b)
                                pltpu.async_copy(table.at[idx2d.at[nli]],
                                                 rows.at[b], sg[b])
                        return carry

                    lax.fori_loop(0, half // K, it, 0)
                    for b in range(K):
                        wait_scatter(b)

        def run_count_pass(cid):
            @pl.when(core == cid)
            def _():
                pltpu.sync_copy(ones_h, rows.at[0])

                def it(t, carry):
                    ds_ = [
                        pltpu.async_copy(rows.at[0],
                                         acc.at[key2d.at[t * K + b]],
                                         ss[b], add=True)
                        for b in range(K)
                    ]
                    for d in ds_:
                        d.wait()
                    return carry

                lax.fori_loop(0, nch // K, it, 0)

        def flush(cid, plane):
            @pl.when(core == cid)
            def _():
                @pl.when(s < 15)
                def _():
                    pltpu.sync_copy(acc.at[pl.ds(s * 1256, 1256)],
                                    out_h.at[plane, pl.ds(s * 1256, 1256)])

                @pl.when(s == 15)
                def _():
                    pltpu.sync_copy(acc.at[pl.ds(18840, 1168)],
                                    out_h.at[plane, pl.ds(18840, 1168)])

        def zero_acc():
            @pl.when(s < 15)
            def _():
                pltpu.sync_copy(zeros_h, acc.at[pl.ds(s * 1256, 1256)])

            @pl.when(s == 15)
            def _():
                pltpu.sync_copy(zeros_h.at[pl.ds(0, 1168)],
                                acc.at[pl.ds(18840, 1168)])

        rounds = [
            ((eq0_h, src_h, 2), (eq1_h, src_h, 3)),
            ((eq2_h, src_h, 4), (eq3_h, src_h, 5)),
            ((rh0_h, brel_h, 0), (rh1_h, brel_h, 1)),
            ((t64_h, time_h, 6), (None, None, 7)),
        ]
        for p0, p1 in rounds:
            zero_acc()
            plsc.subcore_barrier()
            run_pass(0, p0[0], p0[1])
            if p1[0] is None:
                run_count_pass(1)
            else:
                run_pass(1, p1[0], p1[1])
            plsc.subcore_barrier()
            flush(0, p0[2])
            flush(1, p1[2])
            plsc.subcore_barrier()

    return k(eq0, eq1, eq2, eq3, rh0, rh1, t64, ones_t, zeros_t,
             srcR, brelR, timeR, keyR)


def _sc_stage2(ent_new, t64, ones_t, zerosS, zerosT,
               srcR2, dstR2, timeR2, keyR2):
    mesh = plsc.VectorSubcoreMesh(core_axis_name="c", subcore_axis_name="s")
    nch = EP // CH2 // (NC * NS)

    @functools.partial(
        pl.kernel,
        out_type=(
            jax.ShapeDtypeStruct((NC, KB, ENT), F32),
            jax.ShapeDtypeStruct((NC, KB, ENT), F32),
            jax.ShapeDtypeStruct((NC, KB, 64), F32),
            jax.ShapeDtypeStruct((NC, KB, 64), F32),
        ),
        mesh=mesh,
        compiler_params=pltpu.CompilerParams(use_tc_tiling_on_sc=False),
        scratch_types=[
            pltpu.VMEM_SHARED((KB, ENT), F32),
            pltpu.VMEM_SHARED((KB, ENT), F32),
            pltpu.VMEM_SHARED((KB, 64), F32),
            pltpu.VMEM_SHARED((KB, 64), F32),
            pltpu.VMEM((nch, CH2), jnp.int32),
            pltpu.VMEM((nch, CH2), jnp.int32),
            pltpu.VMEM((nch, CH2), jnp.int32),
            pltpu.VMEM((nch, CH2), jnp.int32),
            pltpu.VMEM((2, CH2, ENT), F32),
            pltpu.VMEM((2, CH2, ENT), F32),
            pltpu.VMEM((2, CH2, 64), F32),
            pltpu.VMEM((CH2, 64), F32),
            pltpu.SemaphoreType.DMA,
            pltpu.SemaphoreType.DMA,
            pltpu.SemaphoreType.DMA,
            pltpu.SemaphoreType.DMA,
            pltpu.SemaphoreType.DMA,
            pltpu.SemaphoreType.DMA,
            pltpu.SemaphoreType.DMA,
            pltpu.SemaphoreType.DMA,
            pltpu.SemaphoreType.DMA,
            pltpu.SemaphoreType.DMA,
            pltpu.SemaphoreType.DMA,
            pltpu.SemaphoreType.DMA,
            pltpu.SemaphoreType.DMA,
            pltpu.SemaphoreType.DMA,
        ],
    )
    def k(ent_h, t64_h, ones_h, zS_h, zT_h, src_h, dst_h, time_h, key_h,
          outS_h, outD_h, outT_h, outC_h,
          accS, accD, accT, accC, src2d, dst2d, time2d, key2d,
          rs, rd, rt, rones,
          gs0, gs1, gd0, gd1, gt0, gt1,
          ws0, ws1, wd0, wd1, wt0, wt1, wc0, wc1):
        core = lax.axis_index("c")
        s = lax.axis_index("s")
        gs = [gs0, gs1]
        gd = [gd0, gd1]
        gt = [gt0, gt1]
        ws = [ws0, ws1]
        wd = [wd0, wd1]
        wt = [wt0, wt1]
        wc = [wc0, wc1]
        pltpu.sync_copy(ones_h, rones)
        wid = s * NC + core
        r0 = wid * nch
        pltpu.sync_copy(src_h.at[pl.ds(r0, nch)], src2d)
        pltpu.sync_copy(dst_h.at[pl.ds(r0, nch)], dst2d)
        pltpu.sync_copy(time_h.at[pl.ds(r0, nch)], time2d)
        pltpu.sync_copy(key_h.at[pl.ds(r0, nch)], key2d)

        @pl.when(s == 0)
        def _():
            pltpu.sync_copy(zS_h, accS)

        @pl.when(s == 1)
        def _():
            pltpu.sync_copy(zS_h, accD)

        @pl.when(s == 2)
        def _():
            pltpu.sync_copy(zT_h, accT)

        @pl.when(s == 3)
        def _():
            pltpu.sync_copy(zT_h, accC)

        plsc.subcore_barrier()

        def issue_gathers(i, sl):
            pltpu.async_copy(ent_h.at[src2d.at[i]], rs.at[sl], gs[sl])
            pltpu.async_copy(ent_h.at[dst2d.at[i]], rd.at[sl], gd[sl])
            pltpu.async_copy(t64_h.at[time2d.at[i]], rt.at[sl], gt[sl])

        def wait_gathers(sl):
            pltpu.make_async_copy(ent_h.at[src2d.at[0]], rs.at[sl], gs[sl]).wait()
            pltpu.make_async_copy(ent_h.at[dst2d.at[0]], rd.at[sl], gd[sl]).wait()
            pltpu.make_async_copy(t64_h.at[time2d.at[0]], rt.at[sl], gt[sl]).wait()

        def issue_scatters(i, sl):
            key = key2d.at[i]
            pltpu.async_copy(rs.at[sl], accS.at[key], ws[sl], add=True)
            pltpu.async_copy(rd.at[sl], accD.at[key], wd[sl], add=True)
            pltpu.async_copy(rt.at[sl], accT.at[key], wt[sl], add=True)
            pltpu.async_copy(rones, accC.at[key], wc[sl], add=True)

        def wait_scatters(sl):
            pltpu.make_async_copy(rs.at[sl], accS.at[key2d.at[0]], ws[sl]).wait()
            pltpu.make_async_copy(rd.at[sl], accD.at[key2d.at[0]], wd[sl]).wait()
            pltpu.make_async_copy(rt.at[sl], accT.at[key2d.at[0]], wt[sl]).wait()
            pltpu.make_async_copy(rones, accC.at[key2d.at[0]], wc[sl]).wait()

        issue_gathers(0, 0)
        issue_gathers(1, 1)

        def it(u, carry):
            for sl in range(2):
                i = 2 * u + sl
                wait_gathers(sl)
                issue_scatters(i, sl)
            for sl in range(2):
                ni = 2 * u + 2 + sl

                @pl.when(ni < nch)
                def _(ni=ni, sl=sl):
                    wait_scatters(sl)
                    issue_gathers(ni, sl)
            return carry

        lax.fori_loop(0, nch // 2, it, 0)
        wait_scatters(0)
        wait_scatters(1)
        plsc.subcore_barrier()

        @pl.when(s < 13)
        def _():
            nr = 32
            f0 = s * nr
            pltpu.sync_copy(accS.at[pl.ds(f0, nr)], outS_h.at[core, pl.ds(f0, nr)])
            pltpu.sync_copy(accD.at[pl.ds(f0, nr)], outD_h.at[core, pl.ds(f0, nr)])
            pltpu.sync_copy(accT.at[pl.ds(f0, nr)], outT_h.at[core, pl.ds(f0, nr)])
            pltpu.sync_copy(accC.at[pl.ds(f0, nr)], outC_h.at[core, pl.ds(f0, nr)])

    return k(ent_new, t64, ones_t, zerosS, zerosT, srcR2, dstR2, timeR2, keyR2)


def _tc_stage1(SA, ent_emb, wIt, wOt, wSt, bias3):
    BM = 1000
    nb = N // BM

    def body(s0_ref, s1_ref, e_ref, wI_ref, wO_ref, wS_ref, b_ref, o_ref):
        dot = functools.partial(jnp.dot, preferred_element_type=F32)
        blk0 = s0_ref[...]
        blk1 = s1_ref[...]
        s0 = jnp.concatenate([blk0[p] for p in range(7)], axis=1)
        d0 = blk0[7][:, 0:1]
        s1 = jnp.concatenate([blk1[p] for p in range(7)], axis=1)
        d1 = blk1[7][:, 0:1]
        m = (dot(s0, wI_ref[...]) + d0 * b_ref[0:1, :]
             + dot(s1, wO_ref[...]) + d1 * b_ref[1:2, :])
        h = m / jnp.maximum(d0 + d1, 1.0)
        o_ref[...] = dot(e_ref[...], wS_ref[...]) + b_ref[2:3, :] + h

    return pl.pallas_call(
        body,
        grid=(nb,),
        in_specs=[
            pl.BlockSpec((NPLANES, BM, 64), lambda i: (0, i, 0)),
            pl.BlockSpec((NPLANES, BM, 64), lambda i: (0, i + nb, 0)),
            pl.BlockSpec((BM, ENT), lambda i: (i, 0)),
            pl.BlockSpec((IN_MSG, ENT), lambda i: (0, 0)),
            pl.BlockSpec((IN_MSG, ENT), lambda i: (0, 0)),
            pl.BlockSpec((ENT, ENT), lambda i: (0, 0)),
            pl.BlockSpec((8, ENT), lambda i: (0, 0)),
        ],
        out_specs=pl.BlockSpec((BM, ENT), lambda i: (i, 0)),
        out_shape=jax.ShapeDtypeStruct((N, ENT), F32),
    )(SA, SA, ent_emb, wIt, wOt, wSt, bias3)


def _tc_stage2(outS, outD, outT, outC, rel_emb, wot, wit, wrt, bias3r):
    def body(S_ref, D_ref, T_ref, C_ref, rel_ref, wo_ref, wi_ref, wr_ref,
             b_ref, o_ref):
        dot = functools.partial(jnp.dot, preferred_element_type=F32)
        US = S_ref[0] + S_ref[1]
        UD = D_ref[0] + D_ref[1]
        UT = T_ref[0] + T_ref[1]
        Cc = C_ref[0] + C_ref[1]
        c = Cc[:, 0:1]
        p0 = (dot(US[0:R], wo_ref[0:ENT]) + dot(UD[0:R], wo_ref[ENT:2 * ENT])
              + dot(UT[0:R], wo_ref[2 * ENT:2 * ENT + TIME])
              + c[0:R] * b_ref[0:1, :])
        p1 = (dot(US[R:2 * R], wi_ref[0:ENT])
              + dot(UD[R:2 * R], wi_ref[ENT:2 * ENT])
              + dot(UT[R:2 * R], wi_ref[2 * ENT:2 * ENT + TIME])
              + c[R:2 * R] * b_ref[1:2, :])
        cnt = c[0:R] + c[R:2 * R]
        h = (p0 + p1) / jnp.maximum(cnt, 1.0)
        o_ref[...] = dot(rel_ref[...], wr_ref[...]) + b_ref[2:3, :] + h

    return pl.pallas_call(
        body,
        out_shape=jax.ShapeDtypeStruct((R, REL), F32),
    )(outS, outD, outT, outC, rel_emb, wot, wit, wrt, bias3r)


def kernel(ent_emb, rel_emb, time_emb, edge_index, b_rel, time_idx, inv,
           W_I_w, W_I_b, W_O_w, W_O_b, W_S_w, W_S_b,
           W_r_ori_w, W_r_ori_b, W_r_inv_w, W_r_inv_b, W_R_w, W_R_b):
    i32 = jnp.int32
    pad = EP - E
    src = edge_index[0].astype(i32)
    dst = edge_index[1].astype(i32)
    zpad = jnp.zeros((pad,), i32)
    srcP = jnp.concatenate([src, zpad])
    dstP = jnp.concatenate([dst, zpad])
    brelP = jnp.concatenate([b_rel.astype(i32), zpad])
    timeP = jnp.concatenate([time_idx.astype(i32), zpad])
    invP = jnp.concatenate([inv.astype(i32), jnp.full((pad,), 2, i32)])
    nchT = EP // CH
    nchT2 = EP // CH2
    srcR = srcP.reshape(nchT, CH)
    brelR = brelP.reshape(nchT, CH)
    timeR = timeP.reshape(nchT, CH)
    keyR = (invP * N + dstP).reshape(nchT, CH)
    srcR2 = srcP.reshape(nchT2, CH2)
    dstR2 = dstP.reshape(nchT2, CH2)
    timeR2 = timeP.reshape(nchT2, CH2)
    keyR2 = (invP * R + brelP).reshape(nchT2, CH2)

    eq = [ent_emb[:, 64 * k:64 * (k + 1)] for k in range(4)]
    rh0 = rel_emb[:, :64]
    rh1 = rel_emb[:, 64:]
    ones_t = jnp.ones((CH, 64), F32)
    zeros_t = jnp.zeros((1256, 64), F32)

    SA = _sc_stage1(eq[0], eq[1], eq[2], eq[3], rh0, rh1, time_emb,
                    ones_t, zeros_t, srcR, brelR, timeR, keyR)

    bias3 = jnp.concatenate(
        [W_I_b[None], W_O_b[None], W_S_b[None], jnp.zeros((5, ENT), F32)], 0)
    ent_new = _tc_stage1(SA, ent_emb, W_I_w.T, W_O_w.T, W_S_w.T, bias3)

    ones_t2 = jnp.ones((CH2, 64), F32)
    zerosS = jnp.zeros((KB, ENT), F32)
    zerosT = jnp.zeros((KB, 64), F32)
    outS, outD, outT, outC = _sc_stage2(
        ent_new, time_emb, ones_t2, zerosS, zerosT,
        srcR2, dstR2, timeR2, keyR2)

    bias3r = jnp.concatenate(
        [W_r_ori_b[None], W_r_inv_b[None], W_R_b[None], jnp.zeros((5, REL), F32)], 0)
    rel_new = _tc_stage2(outS, outD, outT, outC, rel_emb,
                         W_r_ori_w.T, W_r_inv_w.T, W_R_w.T, bias3r)
    return ent_new, rel_new
